# baseline (speedup 1.0000x reference)
import functools

import jax
import jax.numpy as jnp
from jax.experimental import pallas as pl
from jax.experimental.pallas import tpu as pltpu

_K = 1024
_D = 256
_C = 192
_B = 32
_BETA = 0.25


def _vq_body(z_ref, e_ref, qst_ref, idx_ref, loss_ref):
    zf = z_ref[0]
    e = e_ref[0]
    z2 = jnp.sum(zf * zf, axis=-1, keepdims=True)
    e2 = jnp.sum(e * e, axis=-1)
    m = jax.lax.dot_general(zf, e, (((1,), (1,)), ((), ())),
                            preferred_element_type=jnp.float32)
    dist = z2 - 2.0 * m + e2[None, :]
    idx = jnp.argmin(dist, axis=-1).astype(jnp.int32)
    idx_ref[0, 0, :] = idx
    oh = (jax.lax.broadcasted_iota(jnp.int32, (_B, _K), 1)
          == idx[:, None]).astype(jnp.float32)
    q = jax.lax.dot_general(oh, e, (((1,), (0,)), ((), ())),
                            preferred_element_type=jnp.float32,
                            precision=jax.lax.Precision.HIGHEST)
    qst_ref[0] = zf + (q - zf)

    c = pl.program_id(0)

    @pl.when(c == 0)
    def _():
        loss_ref[0, 0] = 0.0

    loss_ref[0, 0] += jnp.sum((q - zf) ** 2)

    @pl.when(c == _C - 1)
    def _():
        loss_ref[0, 0] = loss_ref[0, 0] * (_BETA / (_B * _C * _D))


@functools.partial(jax.jit, static_argnames=("interpret",))
def _vq_tc(z_flat, embedding, interpret=False):
    qst, idx3, loss = pl.pallas_call(
        _vq_body,
        grid=(_C,),
        in_specs=[
            pl.BlockSpec((1, _B, _D), lambda c: (c, 0, 0)),
            pl.BlockSpec((1, _K, _D), lambda c: (c, 0, 0)),
        ],
        out_specs=[
            pl.BlockSpec((1, _B, _D), lambda c: (c, 0, 0)),
            pl.BlockSpec((1, 1, _B), lambda c: (c, 0, 0)),
            pl.BlockSpec(memory_space=pltpu.SMEM, block_shape=(1, 1),
                         index_map=lambda c: (0, 0)),
        ],
        out_shape=[
            jax.ShapeDtypeStruct((_C, _B, _D), jnp.float32),
            jax.ShapeDtypeStruct((_C, 1, _B), jnp.int32),
            jax.ShapeDtypeStruct((1, 1), jnp.float32),
        ],
        interpret=interpret,
    )(z_flat, embedding)
    return qst, idx3, loss


def kernel(z, embedding):
    b, c, h, w = z.shape
    d = h * w
    z_flat = z.reshape(b, c, d).transpose(1, 0, 2)
    qst, idx3, loss = _vq_tc(z_flat, embedding)
    q_st = qst.transpose(1, 0, 2).reshape(b, c, h, w)
    indices_out = idx3.reshape(c, b).transpose(1, 0)
    commit_loss = loss[0, 0]
    return (q_st, commit_loss, indices_out)


# speedup vs baseline: 1.4727x; 1.4727x over previous
import functools

import jax
import jax.numpy as jnp
from jax.experimental import pallas as pl
from jax.experimental.pallas import tpu as pltpu

_K = 1024
_D = 256
_C = 192
_B = 32
_BETA = 0.25


_CPB = 4


def _vq_body(z_ref, e_ref, qst_ref, idx_ref, loss_ref):
    step = pl.program_id(0)

    @pl.when(step == 0)
    def _():
        loss_ref[0, 0] = 0.0

    acc = 0.0
    for j in range(_CPB):
        zf = z_ref[j]
        e = e_ref[j]
        z2 = jnp.sum(zf * zf, axis=-1, keepdims=True)
        e2 = jnp.sum(e * e, axis=-1)
        m = jax.lax.dot_general(zf, e, (((1,), (1,)), ((), ())),
                                preferred_element_type=jnp.float32)
        dist = z2 - 2.0 * m + e2[None, :]
        idx = jnp.argmin(dist, axis=-1).astype(jnp.int32)
        idx_ref[j, 0, :] = idx
        oh = (jax.lax.broadcasted_iota(jnp.int32, (_B, _K), 1)
              == idx[:, None]).astype(jnp.float32)
        q = jax.lax.dot_general(oh, e, (((1,), (0,)), ((), ())),
                                preferred_element_type=jnp.float32,
                                precision=jax.lax.Precision.HIGHEST)
        qst_ref[j] = zf + (q - zf)
        acc = acc + jnp.sum((q - zf) ** 2)

    loss_ref[0, 0] += acc

    @pl.when(step == (_C // _CPB) - 1)
    def _():
        loss_ref[0, 0] = loss_ref[0, 0] * (_BETA / (_B * _C * _D))


@functools.partial(jax.jit, static_argnames=("interpret",))
def _vq_tc(z_flat, embedding, interpret=False):
    qst, idx3, loss = pl.pallas_call(
        _vq_body,
        grid=(_C // _CPB,),
        in_specs=[
            pl.BlockSpec((_CPB, _B, _D), lambda c: (c, 0, 0)),
            pl.BlockSpec((_CPB, _K, _D), lambda c: (c, 0, 0)),
        ],
        out_specs=[
            pl.BlockSpec((_CPB, _B, _D), lambda c: (c, 0, 0)),
            pl.BlockSpec((_CPB, 1, _B), lambda c: (c, 0, 0)),
            pl.BlockSpec(memory_space=pltpu.SMEM, block_shape=(1, 1),
                         index_map=lambda c: (0, 0)),
        ],
        out_shape=[
            jax.ShapeDtypeStruct((_C, _B, _D), jnp.float32),
            jax.ShapeDtypeStruct((_C, 1, _B), jnp.int32),
            jax.ShapeDtypeStruct((1, 1), jnp.float32),
        ],
        interpret=interpret,
    )(z_flat, embedding)
    return qst, idx3, loss


def kernel(z, embedding):
    b, c, h, w = z.shape
    d = h * w
    z_flat = z.reshape(b, c, d).transpose(1, 0, 2)
    qst, idx3, loss = _vq_tc(z_flat, embedding)
    q_st = qst.transpose(1, 0, 2).reshape(b, c, h, w)
    indices_out = idx3.reshape(c, b).transpose(1, 0)
    commit_loss = loss[0, 0]
    return (q_st, commit_loss, indices_out)


# speedup vs baseline: 1.5134x; 1.0277x over previous
import functools

import jax
import jax.numpy as jnp
from jax.experimental import pallas as pl
from jax.experimental.pallas import tpu as pltpu

_K = 1024
_D = 256
_C = 192
_B = 32
_BETA = 0.25


_CPB = 8


def _vq_body(z_ref, e_ref, qst_ref, idx_ref, loss_ref):
    step = pl.program_id(0)

    @pl.when(step == 0)
    def _():
        loss_ref[0, 0] = 0.0

    acc = 0.0
    for j in range(_CPB):
        zf = z_ref[j]
        e = e_ref[j]
        z2 = jnp.sum(zf * zf, axis=-1, keepdims=True)
        e2 = jnp.sum(e * e, axis=-1)
        m = jax.lax.dot_general(zf, e, (((1,), (1,)), ((), ())),
                                preferred_element_type=jnp.float32)
        dist = z2 - 2.0 * m + e2[None, :]
        idx = jnp.argmin(dist, axis=-1).astype(jnp.int32)
        idx_ref[j, 0, :] = idx
        oh = (jax.lax.broadcasted_iota(jnp.int32, (_B, _K), 1)
              == idx[:, None]).astype(jnp.float32)
        q = jax.lax.dot_general(oh, e, (((1,), (0,)), ((), ())),
                                preferred_element_type=jnp.float32,
                                precision=jax.lax.Precision.HIGHEST)
        qst_ref[j] = zf + (q - zf)
        acc = acc + jnp.sum((q - zf) ** 2)

    loss_ref[0, 0] += acc

    @pl.when(step == (_C // _CPB) - 1)
    def _():
        loss_ref[0, 0] = loss_ref[0, 0] * (_BETA / (_B * _C * _D))


@functools.partial(jax.jit, static_argnames=("interpret",))
def _vq_tc(z_flat, embedding, interpret=False):
    qst, idx3, loss = pl.pallas_call(
        _vq_body,
        grid=(_C // _CPB,),
        in_specs=[
            pl.BlockSpec((_CPB, _B, _D), lambda c: (c, 0, 0)),
            pl.BlockSpec((_CPB, _K, _D), lambda c: (c, 0, 0)),
        ],
        out_specs=[
            pl.BlockSpec((_CPB, _B, _D), lambda c: (c, 0, 0)),
            pl.BlockSpec((_CPB, 1, _B), lambda c: (c, 0, 0)),
            pl.BlockSpec(memory_space=pltpu.SMEM, block_shape=(1, 1),
                         index_map=lambda c: (0, 0)),
        ],
        out_shape=[
            jax.ShapeDtypeStruct((_C, _B, _D), jnp.float32),
            jax.ShapeDtypeStruct((_C, 1, _B), jnp.int32),
            jax.ShapeDtypeStruct((1, 1), jnp.float32),
        ],
        interpret=interpret,
    )(z_flat, embedding)
    return qst, idx3, loss


def kernel(z, embedding):
    b, c, h, w = z.shape
    d = h * w
    z_flat = z.reshape(b, c, d).transpose(1, 0, 2)
    qst, idx3, loss = _vq_tc(z_flat, embedding)
    q_st = qst.transpose(1, 0, 2).reshape(b, c, h, w)
    indices_out = idx3.reshape(c, b).transpose(1, 0)
    commit_loss = loss[0, 0]
    return (q_st, commit_loss, indices_out)


# speedup vs baseline: 2.0797x; 1.3741x over previous
import functools

import jax
import jax.numpy as jnp
from jax import lax
from jax.experimental import pallas as pl
from jax.experimental.pallas import tpu as pltpu
from jax.experimental.pallas import tpu_sc as plsc

_K = 1024
_D = 256
_C = 192
_B = 32
_BETA = 0.25

_CPB = 8


def _vq_body(z_ref, e_ref, idx_ref, flat_ref, loss_ref):
    step = pl.program_id(0)

    @pl.when(step == 0)
    def _():
        loss_ref[0, 0] = 0.0

    acc = 0.0
    for j in range(_CPB):
        zf = z_ref[j]
        e = e_ref[j]
        z2 = jnp.sum(zf * zf, axis=-1, keepdims=True)
        e2 = jnp.sum(e * e, axis=-1)
        m = jax.lax.dot_general(zf, e, (((1,), (1,)), ((), ())),
                                preferred_element_type=jnp.float32)
        dist = z2 - 2.0 * m + e2[None, :]
        idx = jnp.argmin(dist, axis=-1).astype(jnp.int32)
        idx_ref[j, 0, :] = idx
        flat_ref[j, 0, :] = idx + (step * _CPB + j) * _K
        acc = acc + jnp.sum(jnp.min(dist, axis=-1))

    loss_ref[0, 0] += acc

    @pl.when(step == (_C // _CPB) - 1)
    def _():
        loss_ref[0, 0] = loss_ref[0, 0] * (_BETA / (_B * _C * _D))


@functools.partial(jax.jit, static_argnames=("interpret",))
def _vq_tc(z_flat, embedding, interpret=False):
    idx3, flat3, loss = pl.pallas_call(
        _vq_body,
        grid=(_C // _CPB,),
        in_specs=[
            pl.BlockSpec((_CPB, _B, _D), lambda c: (c, 0, 0)),
            pl.BlockSpec((_CPB, _K, _D), lambda c: (c, 0, 0)),
        ],
        out_specs=[
            pl.BlockSpec((_CPB, 1, _B), lambda c: (c, 0, 0)),
            pl.BlockSpec((_CPB, 1, _B), lambda c: (c, 0, 0)),
            pl.BlockSpec(memory_space=pltpu.SMEM, block_shape=(1, 1),
                         index_map=lambda c: (0, 0)),
        ],
        out_shape=[
            jax.ShapeDtypeStruct((_C, 1, _B), jnp.int32),
            jax.ShapeDtypeStruct((_C, 1, _B), jnp.int32),
            jax.ShapeDtypeStruct((1, 1), jnp.float32),
        ],
        interpret=interpret,
    )(z_flat, embedding)
    return idx3, flat3, loss


def _make_sc_gather():
    info = plsc.get_sparse_core_info()
    nw = info.num_cores * info.num_subcores
    rows = _B * _C
    rpw = rows // nw
    nch = 2
    cpw = rpw // nch
    mesh = plsc.VectorSubcoreMesh(core_axis_name="c", subcore_axis_name="s")

    @functools.partial(
        pl.kernel, mesh=mesh,
        out_type=jax.ShapeDtypeStruct((rows, _D), jnp.float32),
        scratch_types=[
            pltpu.VMEM((nch, cpw), jnp.int32),
            pltpu.VMEM((cpw, _D), jnp.float32),
            pltpu.VMEM((cpw, _D), jnp.float32),
            pltpu.SemaphoreType.DMA,
            pltpu.SemaphoreType.DMA,
        ],
    )
    def sc_gather(table_hbm, idx_hbm, out_hbm, idx_v, rows_a, rows_b, sem_a, sem_b):
        wid = lax.axis_index("s") * info.num_cores + lax.axis_index("c")
        pltpu.sync_copy(idx_hbm.at[wid], idx_v)
        base = wid * rpw
        cp_a = pltpu.async_copy(table_hbm.at[idx_v.at[0]], rows_a, sem_a)
        cp_b = pltpu.async_copy(table_hbm.at[idx_v.at[1]], rows_b, sem_b)
        cp_a.wait()
        pltpu.sync_copy(rows_a, out_hbm.at[pl.ds(base, cpw)])
        cp_b.wait()
        pltpu.sync_copy(rows_b, out_hbm.at[pl.ds(base + cpw, cpw)])

    return sc_gather, nw, nch, cpw


def kernel(z, embedding):
    b, c, h, w = z.shape
    d = h * w
    z_flat = z.reshape(b, c, d).transpose(1, 0, 2)
    idx3, flat3, loss = _vq_tc(z_flat, embedding)
    indices_out = idx3.reshape(c, b).transpose(1, 0)
    commit_loss = loss[0, 0]

    sc_gather, nw, nch, cpw = _make_sc_gather()
    table = embedding.reshape(c * _K, d)
    flat_idx = flat3.reshape(c, b).transpose(1, 0).reshape(nw, nch, cpw)
    q_rows = sc_gather(table, flat_idx)
    q_st = q_rows.reshape(b, c, h, w)
    return (q_st, commit_loss, indices_out)


# speedup vs baseline: 2.0983x; 1.0090x over previous
import functools

import jax
import jax.numpy as jnp
from jax import lax
from jax.experimental import pallas as pl
from jax.experimental.pallas import tpu as pltpu
from jax.experimental.pallas import tpu_sc as plsc

_K = 1024
_D = 256
_C = 192
_B = 32
_BETA = 0.25

_CPB = 8


def _vq_body(z_ref, e_ref, idx_ref, flat_ref, loss_ref):
    step = pl.program_id(0)

    @pl.when(step == 0)
    def _():
        loss_ref[0, 0] = 0.0

    acc = 0.0
    for j in range(_CPB):
        zf = z_ref[:, j, :]
        e = e_ref[j]
        z2 = jnp.sum(zf * zf, axis=-1, keepdims=True)
        e2 = jnp.sum(e * e, axis=-1)
        m = jax.lax.dot_general(zf, e, (((1,), (1,)), ((), ())),
                                preferred_element_type=jnp.float32)
        dist = z2 - 2.0 * m + e2[None, :]
        idx = jnp.argmin(dist, axis=-1).astype(jnp.int32)
        idx_ref[j, 0, :] = idx
        flat_ref[j, 0, :] = idx + (step * _CPB + j) * _K
        acc = acc + jnp.sum(jnp.min(dist, axis=-1))

    loss_ref[0, 0] += acc

    @pl.when(step == (_C // _CPB) - 1)
    def _():
        loss_ref[0, 0] = loss_ref[0, 0] * (_BETA / (_B * _C * _D))


@functools.partial(jax.jit, static_argnames=("interpret",))
def _vq_tc(z_flat, embedding, interpret=False):
    idx3, flat3, loss = pl.pallas_call(
        _vq_body,
        grid=(_C // _CPB,),
        in_specs=[
            pl.BlockSpec((_B, _CPB, _D), lambda c: (0, c, 0)),
            pl.BlockSpec((_CPB, _K, _D), lambda c: (c, 0, 0)),
        ],
        out_specs=[
            pl.BlockSpec((_CPB, 1, _B), lambda c: (c, 0, 0)),
            pl.BlockSpec((_CPB, 1, _B), lambda c: (c, 0, 0)),
            pl.BlockSpec(memory_space=pltpu.SMEM, block_shape=(1, 1),
                         index_map=lambda c: (0, 0)),
        ],
        out_shape=[
            jax.ShapeDtypeStruct((_C, 1, _B), jnp.int32),
            jax.ShapeDtypeStruct((_C, 1, _B), jnp.int32),
            jax.ShapeDtypeStruct((1, 1), jnp.float32),
        ],
        interpret=interpret,
    )(z_flat, embedding)
    return idx3, flat3, loss


def _make_sc_gather():
    info = plsc.get_sparse_core_info()
    nw = info.num_cores * info.num_subcores
    rows = _B * _C
    rpw = rows // nw
    nch = 2
    cpw = rpw // nch
    mesh = plsc.VectorSubcoreMesh(core_axis_name="c", subcore_axis_name="s")

    @functools.partial(
        pl.kernel, mesh=mesh,
        out_type=jax.ShapeDtypeStruct((rows, _D), jnp.float32),
        scratch_types=[
            pltpu.VMEM((nch, cpw), jnp.int32),
            pltpu.VMEM((cpw, _D), jnp.float32),
            pltpu.VMEM((cpw, _D), jnp.float32),
            pltpu.SemaphoreType.DMA,
            pltpu.SemaphoreType.DMA,
        ],
    )
    def sc_gather(table_hbm, idx_hbm, out_hbm, idx_v, rows_a, rows_b, sem_a, sem_b):
        wid = lax.axis_index("s") * info.num_cores + lax.axis_index("c")
        pltpu.sync_copy(idx_hbm.at[wid], idx_v)
        base = wid * rpw
        cp_a = pltpu.async_copy(table_hbm.at[idx_v.at[0]], rows_a, sem_a)
        cp_b = pltpu.async_copy(table_hbm.at[idx_v.at[1]], rows_b, sem_b)
        cp_a.wait()
        pltpu.sync_copy(rows_a, out_hbm.at[pl.ds(base, cpw)])
        cp_b.wait()
        pltpu.sync_copy(rows_b, out_hbm.at[pl.ds(base + cpw, cpw)])

    return sc_gather, nw, nch, cpw


def kernel(z, embedding):
    b, c, h, w = z.shape
    d = h * w
    z_flat = z.reshape(b, c, d)
    idx3, flat3, loss = _vq_tc(z_flat, embedding)
    indices_out = idx3.reshape(c, b).transpose(1, 0)
    commit_loss = loss[0, 0]

    sc_gather, nw, nch, cpw = _make_sc_gather()
    table = embedding.reshape(c * _K, d)
    flat_idx = flat3.reshape(c, b).transpose(1, 0).reshape(nw, nch, cpw)
    q_rows = sc_gather(table, flat_idx)
    q_st = q_rows.reshape(b, c, h, w)
    return (q_st, commit_loss, indices_out)
